# Initial kernel scaffold; baseline (speedup 1.0000x reference)
import functools

import jax
import jax.numpy as jnp
from jax import lax
from jax.experimental import pallas as pl
from jax.experimental.pallas import tpu as pltpu
from jax.experimental.pallas import tpu_sc as plsc

_N = 100
_STATES = _N ** 3 + 1
_GAMMA = 0.9
_ALPHA = 0.1
_B = 16384

_NC, _NS, _L = 2, 16, 16
_NW = _NC * _NS
_BPW = _B // _NW
_CH = 128
_NCHUNK = _BPW // _CH
_NG = _CH // _L

_mesh = plsc.VectorSubcoreMesh(core_axis_name="c", subcore_axis_name="s")


@functools.partial(
    pl.kernel,
    out_type=(),
    mesh=_mesh,
    scratch_types=[
        pltpu.VMEM((_NCHUNK, _CH), jnp.int32),
        pltpu.VMEM((_NCHUNK, _CH), jnp.int32),
        pltpu.VMEM((_NCHUNK, _CH), jnp.int32),
        pltpu.VMEM((_NCHUNK, _CH), jnp.int32),
        pltpu.VMEM((_NCHUNK, _CH), jnp.float32),
        pltpu.VMEM((_NCHUNK, _CH), jnp.float32),
        pltpu.VMEM((_NCHUNK, _CH), jnp.float32),
        pltpu.VMEM((_NCHUNK, _CH, _N), jnp.float32),
        pltpu.SemaphoreType.DMA,
        pltpu.SemaphoreType.DMA,
    ],
)
def _sc_update(q2d, qflat, prevr, actr, nxtr, rewr, qout,
               nidx_v, pv_v, av_v, fidx_v, rew_v, old_v, new_v, rows_v,
               gsem, ssem):
    wid = lax.axis_index("s") * _NC + lax.axis_index("c")

    pltpu.sync_copy(nxtr.at[wid], nidx_v)
    pltpu.sync_copy(prevr.at[wid], pv_v)
    pltpu.sync_copy(actr.at[wid], av_v)
    pltpu.sync_copy(rewr.at[wid], rew_v)

    row_cps = [
        pltpu.async_copy(q2d.at[nidx_v.at[j]], rows_v.at[j], gsem)
        for j in range(_NCHUNK)
    ]

    for j in range(_NCHUNK):
        for g in range(_NG):
            sl = pl.ds(g * _L, _L)
            fidx_v[j, sl] = pv_v[j, sl] * _N + av_v[j, sl]

    old_cps = [
        pltpu.async_copy(qflat.at[fidx_v.at[j]], old_v.at[j], gsem)
        for j in range(_NCHUNK)
    ]

    scat_cps = []
    for j in range(_NCHUNK):
        row_cps[j].wait()
        old_cps[j].wait()
        rows_j = rows_v.at[j]
        for g in range(_NG):
            sl = pl.ds(g * _L, _L)
            rowids = g * _L + lax.iota(jnp.int32, _L)

            def body(jj, acc, rowids=rowids, rows_j=rows_j):
                colids = jnp.full((_L,), jj, jnp.int32)
                vals = plsc.load_gather(rows_j, [rowids, colids])
                return jnp.maximum(acc, vals)

            v = lax.fori_loop(
                0, _N, body, jnp.full((_L,), -jnp.inf, jnp.float32))
            target = rew_v[j, sl] + _GAMMA * v
            new_v[j, sl] = (1.0 - _ALPHA) * old_v[j, sl] + _ALPHA * target
        scat_cps.append(
            pltpu.async_copy(new_v.at[j], qout.at[fidx_v.at[j]], ssem))
    for cp in scat_cps:
        cp.wait()


def _copy_body(src_ref, dst_ref):
    dst_ref[...] = src_ref[...]


_ROWS_BLK = 8192
_copy_q = pl.pallas_call(
    _copy_body,
    grid=(pl.cdiv(_STATES, _ROWS_BLK),),
    in_specs=[pl.BlockSpec((_ROWS_BLK, _N), lambda i: (i, 0))],
    out_specs=pl.BlockSpec((_ROWS_BLK, _N), lambda i: (i, 0)),
    out_shape=jax.ShapeDtypeStruct((_STATES, _N), jnp.float32),
)


def kernel(Q, prev_state_idx, action, next_state_idx, reward):
    qcopy = _copy_q(Q)
    qref = jax.new_ref(qcopy.reshape(_STATES * _N))
    _sc_update(
        Q,
        Q.reshape(_STATES * _N),
        prev_state_idx.reshape(_NW, _NCHUNK, _CH),
        action.reshape(_NW, _NCHUNK, _CH),
        next_state_idx.reshape(_NW, _NCHUNK, _CH),
        reward.reshape(_NW, _NCHUNK, _CH),
        qref,
    )
    return qref[...].reshape(_STATES, _N)


# speedup vs baseline: 12.4594x; 12.4594x over previous
import functools

import jax
import jax.numpy as jnp
from jax import lax
from jax.experimental import pallas as pl
from jax.experimental.pallas import tpu as pltpu
from jax.experimental.pallas import tpu_sc as plsc

_N = 100
_STATES = _N ** 3 + 1
_GAMMA = 0.9
_ALPHA = 0.1
_B = 16384

_PITCH = 128
_RPAD = ((_STATES + 7) // 8) * 8
_QPAD = _RPAD * _PITCH

_NC, _NS, _L = 2, 16, 16
_NW = _NC * _NS
_BPW = _B // _NW
_CH = 128
_NCHUNK = _BPW // _CH

_BLK = 8192
_GRID = pl.cdiv(_STATES, _BLK)

_mesh = plsc.VectorSubcoreMesh(core_axis_name="c", subcore_axis_name="s")


def _prep_body(src_ref, qpad_ref, rowmax_ref):
    x = src_ref[...]
    rowmax_ref[...] = jnp.max(x, axis=1)
    y = jnp.concatenate(
        [x, jnp.zeros((_BLK, _PITCH - _N), jnp.float32)], axis=1)
    qpad_ref[...] = y.reshape(_BLK * _PITCH)


_prep = pl.pallas_call(
    _prep_body,
    grid=(_GRID,),
    in_specs=[pl.BlockSpec((_BLK, _N), lambda i: (i, 0))],
    out_specs=[
        pl.BlockSpec((_BLK * _PITCH,), lambda i: (i,)),
        pl.BlockSpec((_BLK,), lambda i: (i,)),
    ],
    out_shape=[
        jax.ShapeDtypeStruct((_QPAD,), jnp.float32),
        jax.ShapeDtypeStruct((_STATES,), jnp.float32),
    ],
)


@functools.partial(
    pl.kernel,
    out_type=(),
    mesh=_mesh,
    compiler_params=pltpu.CompilerParams(needs_layout_passes=False),
    scratch_types=[
        pltpu.VMEM((_BPW,), jnp.int32),
        pltpu.VMEM((_BPW,), jnp.int32),
        pltpu.VMEM((_BPW,), jnp.int32),
        pltpu.VMEM((_BPW,), jnp.float32),
        pltpu.VMEM((_NCHUNK, _CH), jnp.int32),
        pltpu.VMEM((_NCHUNK, _CH), jnp.int32),
        pltpu.VMEM((_NCHUNK, _CH), jnp.float32),
        pltpu.VMEM((_NCHUNK, _CH), jnp.float32),
        pltpu.VMEM((_NCHUNK, _CH), jnp.float32),
        pltpu.SemaphoreType.DMA,
        pltpu.SemaphoreType.DMA,
    ],
)
def _sc_update(rowmax, prevs, acts, nxts, rews, qpad,
               pv, av, nv, rw, fidx, nidx, old, vmx, newv, gsem, ssem):
    wid = lax.axis_index("s") * _NC + lax.axis_index("c")
    base = wid * _BPW

    pltpu.sync_copy(prevs.at[pl.ds(base, _BPW)], pv)
    pltpu.sync_copy(acts.at[pl.ds(base, _BPW)], av)
    pltpu.sync_copy(nxts.at[pl.ds(base, _BPW)], nv)
    pltpu.sync_copy(rews.at[pl.ds(base, _BPW)], rw)

    for k in range(_BPW // _L):
        j, sl = k // (_CH // _L), pl.ds((k % (_CH // _L)) * _L, _L)
        s16 = pl.ds(k * _L, _L)
        fidx[j, sl] = pv[s16] * _PITCH + av[s16]
        nidx[j, sl] = nv[s16]

    cps = []
    for j in range(_NCHUNK):
        cps.append(pltpu.async_copy(qpad.at[fidx.at[j]], old.at[j], gsem))
        cps.append(pltpu.async_copy(rowmax.at[nidx.at[j]], vmx.at[j], gsem))
    for cp in cps:
        cp.wait()

    for k in range(_BPW // _L):
        j, sl = k // (_CH // _L), pl.ds((k % (_CH // _L)) * _L, _L)
        s16 = pl.ds(k * _L, _L)
        target = rw[s16] + _GAMMA * vmx[j, sl]
        newv[j, sl] = (1.0 - _ALPHA) * old[j, sl] + _ALPHA * target

    scs = [
        pltpu.async_copy(newv.at[j], qpad.at[fidx.at[j]], ssem)
        for j in range(_NCHUNK)
    ]
    for cp in scs:
        cp.wait()


def _depad_body(qpad_ref, dst_ref):
    y = qpad_ref[...].reshape(_BLK, _PITCH)
    dst_ref[...] = y[:, :_N]


_depad = pl.pallas_call(
    _depad_body,
    grid=(_GRID,),
    in_specs=[pl.BlockSpec((_BLK * _PITCH,), lambda i: (i,))],
    out_specs=pl.BlockSpec((_BLK, _N), lambda i: (i, 0)),
    out_shape=jax.ShapeDtypeStruct((_STATES, _N), jnp.float32),
)


def kernel(Q, prev_state_idx, action, next_state_idx, reward):
    qpad, rowmax = _prep(Q)
    qref = jax.new_ref(qpad)
    _sc_update(rowmax, prev_state_idx, action, next_state_idx, reward, qref)
    return _depad(qref[...])


# speedup vs baseline: 31.5820x; 2.5348x over previous
import functools

import jax
import jax.numpy as jnp
from jax import lax
from jax.experimental import pallas as pl
from jax.experimental.pallas import tpu as pltpu
from jax.experimental.pallas import tpu_sc as plsc

_N = 100
_STATES = _N ** 3 + 1
_GAMMA = 0.9
_ALPHA = 0.1
_B = 16384

_PITCH = 128
_RPAD = ((_STATES + 7) // 8) * 8
_QPAD = _RPAD * _PITCH

_NC, _NS, _L = 2, 16, 16
_NW = _NC * _NS
_BPW = _B // _NW
_CH = 128
_NCHUNK = _BPW // _CH

_BLK = 8192
_GRID = pl.cdiv(_STATES, _BLK)

_mesh = plsc.VectorSubcoreMesh(core_axis_name="c", subcore_axis_name="s")


def _prep_body(src_ref, qpad_ref, rowmax_ref):
    x = src_ref[...]
    rowmax_ref[...] = jnp.max(x, axis=0)
    xp = jnp.concatenate(
        [x, jnp.zeros((_PITCH - _N, _BLK), jnp.float32)], axis=0)
    qpad_ref[...] = xp.T.reshape(_BLK * _PITCH)


_prep = pl.pallas_call(
    _prep_body,
    grid=(_GRID,),
    in_specs=[pl.BlockSpec((_N, _BLK), lambda i: (0, i))],
    out_specs=[
        pl.BlockSpec((_BLK * _PITCH,), lambda i: (i,)),
        pl.BlockSpec((_BLK,), lambda i: (i,)),
    ],
    out_shape=[
        jax.ShapeDtypeStruct((_QPAD,), jnp.float32),
        jax.ShapeDtypeStruct((_STATES,), jnp.float32),
    ],
)


@functools.partial(
    pl.kernel,
    out_type=(),
    mesh=_mesh,
    compiler_params=pltpu.CompilerParams(needs_layout_passes=False),
    scratch_types=[
        pltpu.VMEM((_BPW,), jnp.int32),
        pltpu.VMEM((_BPW,), jnp.int32),
        pltpu.VMEM((_BPW,), jnp.int32),
        pltpu.VMEM((_BPW,), jnp.float32),
        pltpu.VMEM((_NCHUNK, _CH), jnp.int32),
        pltpu.VMEM((_NCHUNK, _CH), jnp.int32),
        pltpu.VMEM((_NCHUNK, _CH), jnp.float32),
        pltpu.VMEM((_NCHUNK, _CH), jnp.float32),
        pltpu.VMEM((_NCHUNK, _CH), jnp.float32),
        pltpu.SemaphoreType.DMA,
        pltpu.SemaphoreType.DMA,
    ],
)
def _sc_update(rowmax, prevs, acts, nxts, rews, qpad,
               pv, av, nv, rw, fidx, nidx, old, vmx, newv, gsem, ssem):
    wid = lax.axis_index("s") * _NC + lax.axis_index("c")
    base = wid * _BPW

    pltpu.sync_copy(prevs.at[pl.ds(base, _BPW)], pv)
    pltpu.sync_copy(acts.at[pl.ds(base, _BPW)], av)
    pltpu.sync_copy(nxts.at[pl.ds(base, _BPW)], nv)
    pltpu.sync_copy(rews.at[pl.ds(base, _BPW)], rw)

    for k in range(_BPW // _L):
        j, sl = k // (_CH // _L), pl.ds((k % (_CH // _L)) * _L, _L)
        s16 = pl.ds(k * _L, _L)
        fidx[j, sl] = pv[s16] * _PITCH + av[s16]
        nidx[j, sl] = nv[s16]

    cps = []
    for j in range(_NCHUNK):
        cps.append(pltpu.async_copy(qpad.at[fidx.at[j]], old.at[j], gsem))
        cps.append(pltpu.async_copy(rowmax.at[nidx.at[j]], vmx.at[j], gsem))
    for cp in cps:
        cp.wait()

    for k in range(_BPW // _L):
        j, sl = k // (_CH // _L), pl.ds((k % (_CH // _L)) * _L, _L)
        s16 = pl.ds(k * _L, _L)
        target = rw[s16] + _GAMMA * vmx[j, sl]
        newv[j, sl] = (1.0 - _ALPHA) * old[j, sl] + _ALPHA * target

    scs = [
        pltpu.async_copy(newv.at[j], qpad.at[fidx.at[j]], ssem)
        for j in range(_NCHUNK)
    ]
    for cp in scs:
        cp.wait()


def _depad_body(qpad_ref, dst_ref):
    y = qpad_ref[...].reshape(_BLK, _PITCH)
    dst_ref[...] = y.T[:_N, :]


_depad = pl.pallas_call(
    _depad_body,
    grid=(_GRID,),
    in_specs=[pl.BlockSpec((_BLK * _PITCH,), lambda i: (i,))],
    out_specs=pl.BlockSpec((_N, _BLK), lambda i: (0, i)),
    out_shape=jax.ShapeDtypeStruct((_N, _STATES), jnp.float32),
)


def kernel(Q, prev_state_idx, action, next_state_idx, reward):
    qpad, rowmax = _prep(Q.T)
    qref = jax.new_ref(qpad)
    _sc_update(rowmax, prev_state_idx, action, next_state_idx, reward, qref)
    return _depad(qref[...]).T


# speedup vs baseline: 32.6701x; 1.0345x over previous
import functools

import jax
import jax.numpy as jnp
from jax import lax
from jax.experimental import pallas as pl
from jax.experimental.pallas import tpu as pltpu
from jax.experimental.pallas import tpu_sc as plsc

_N = 100
_STATES = _N ** 3 + 1
_GAMMA = 0.9
_ALPHA = 0.1
_B = 16384

_PITCH = 128
_RPAD = ((_STATES + 7) // 8) * 8
_QPAD = _RPAD * _PITCH

_NC, _NS, _L = 2, 16, 16
_NW = _NC * _NS
_BPW = _B // _NW
_CH = 128
_NCHUNK = _BPW // _CH

_BLK = 16384
_GRID = pl.cdiv(_STATES, _BLK)

_mesh = plsc.VectorSubcoreMesh(core_axis_name="c", subcore_axis_name="s")


def _prep_body(src_ref, qpad_ref, rowmax_ref):
    x = src_ref[...]
    rowmax_ref[...] = jnp.max(x, axis=0)
    xp = jnp.concatenate(
        [x, jnp.zeros((_PITCH - _N, _BLK), jnp.float32)], axis=0)
    qpad_ref[...] = xp.T.reshape(_BLK * _PITCH)


_prep = pl.pallas_call(
    _prep_body,
    grid=(_GRID,),
    in_specs=[pl.BlockSpec((_N, _BLK), lambda i: (0, i))],
    out_specs=[
        pl.BlockSpec((_BLK * _PITCH,), lambda i: (i,)),
        pl.BlockSpec((_BLK,), lambda i: (i,)),
    ],
    out_shape=[
        jax.ShapeDtypeStruct((_QPAD,), jnp.float32),
        jax.ShapeDtypeStruct((_STATES,), jnp.float32),
    ],
)


@functools.partial(
    pl.kernel,
    out_type=(),
    mesh=_mesh,
    compiler_params=pltpu.CompilerParams(needs_layout_passes=False),
    scratch_types=[
        pltpu.VMEM((_BPW,), jnp.int32),
        pltpu.VMEM((_BPW,), jnp.int32),
        pltpu.VMEM((_BPW,), jnp.int32),
        pltpu.VMEM((_BPW,), jnp.float32),
        pltpu.VMEM((_NCHUNK, _CH), jnp.int32),
        pltpu.VMEM((_NCHUNK, _CH), jnp.int32),
        pltpu.VMEM((_NCHUNK, _CH), jnp.float32),
        pltpu.VMEM((_NCHUNK, _CH), jnp.float32),
        pltpu.VMEM((_NCHUNK, _CH), jnp.float32),
        pltpu.SemaphoreType.DMA,
        pltpu.SemaphoreType.DMA,
    ],
)
def _sc_update(rowmax, prevs, acts, nxts, rews, qpad,
               pv, av, nv, rw, fidx, nidx, old, vmx, newv, gsem, ssem):
    wid = lax.axis_index("s") * _NC + lax.axis_index("c")
    base = wid * _BPW

    pltpu.sync_copy(prevs.at[pl.ds(base, _BPW)], pv)
    pltpu.sync_copy(acts.at[pl.ds(base, _BPW)], av)
    pltpu.sync_copy(nxts.at[pl.ds(base, _BPW)], nv)
    pltpu.sync_copy(rews.at[pl.ds(base, _BPW)], rw)

    for k in range(_BPW // _L):
        j, sl = k // (_CH // _L), pl.ds((k % (_CH // _L)) * _L, _L)
        s16 = pl.ds(k * _L, _L)
        fidx[j, sl] = pv[s16] * _PITCH + av[s16]
        nidx[j, sl] = nv[s16]

    cps = []
    for j in range(_NCHUNK):
        cps.append(pltpu.async_copy(qpad.at[fidx.at[j]], old.at[j], gsem))
        cps.append(pltpu.async_copy(rowmax.at[nidx.at[j]], vmx.at[j], gsem))
    for cp in cps:
        cp.wait()

    for k in range(_BPW // _L):
        j, sl = k // (_CH // _L), pl.ds((k % (_CH // _L)) * _L, _L)
        s16 = pl.ds(k * _L, _L)
        target = rw[s16] + _GAMMA * vmx[j, sl]
        newv[j, sl] = (1.0 - _ALPHA) * old[j, sl] + _ALPHA * target

    scs = [
        pltpu.async_copy(newv.at[j], qpad.at[fidx.at[j]], ssem)
        for j in range(_NCHUNK)
    ]
    for cp in scs:
        cp.wait()


def _depad_body(qpad_ref, dst_ref):
    y = qpad_ref[...].reshape(_BLK, _PITCH)
    dst_ref[...] = y.T[:_N, :]


_depad = pl.pallas_call(
    _depad_body,
    grid=(_GRID,),
    in_specs=[pl.BlockSpec((_BLK * _PITCH,), lambda i: (i,))],
    out_specs=pl.BlockSpec((_N, _BLK), lambda i: (0, i)),
    out_shape=jax.ShapeDtypeStruct((_N, _STATES), jnp.float32),
)


def kernel(Q, prev_state_idx, action, next_state_idx, reward):
    qpad, rowmax = _prep(Q.T)
    qref = jax.new_ref(qpad)
    _sc_update(rowmax, prev_state_idx, action, next_state_idx, reward, qref)
    return _depad(qref[...]).T
